# Initial kernel scaffold; baseline (speedup 1.0000x reference)
import jax
import jax.numpy as jnp
from jax.experimental import pallas as pl


def kernel(x, edge_index, Ws1, Wn1, bc1, Ws2, Wn2, bc2, Ws3, Wn3, bc3, gamma1, beta1, gamma2, beta2, gamma3, beta3, fc1_W, fc1_b, fc2_W, fc2_b, fc3_W, fc3_b):
    raise NotImplementedError("write your pallas kernel here")


# speedup vs baseline: 2.6716x; 2.6716x over previous
import functools

import jax
import jax.numpy as jnp
from jax import lax
from jax.experimental import pallas as pl
from jax.experimental.pallas import tpu as pltpu
from jax.experimental.pallas import tpu_sc as plsc

N = 50000
E = 800000
HID = 256
N_CLASSES = 18
EPS = 1e-5
SLOPE = 0.01

EROWS = 6272
ROWS_PER_TILE = EROWS // 16
IDX_BLK = 8
N_OUTER = ROWS_PER_TILE // IDX_BLK
NPAD = N + 8
NACC = 50048
RPT_NODES = NACC // 16
ZROWS = 184


def _make_sc_agg(n_chunks):
  half = n_chunks // 2
  mesh = plsc.VectorSubcoreMesh(core_axis_name="c", subcore_axis_name="s")

  @functools.partial(
      pl.kernel,
      mesh=mesh,
      out_type=jax.ShapeDtypeStruct((n_chunks, NACC, 32), jnp.float32),
      scratch_types=[
          pltpu.VMEM_SHARED((NACC, 32), jnp.float32),
          pltpu.VMEM((ZROWS, 32), jnp.float32),
          pltpu.VMEM((IDX_BLK, 128), jnp.int32),
          pltpu.VMEM((IDX_BLK, 128), jnp.int32),
          pltpu.VMEM((128, 32), jnp.float32),
          pltpu.SemaphoreType.DMA,
      ],
      compiler_params=pltpu.CompilerParams(use_tc_tiling_on_sc=False),
  )
  def k(h_hbm, src_hbm, dst_hbm, out_hbm, agg_s, zbuf, srcb, dstb, rows, gsem):
    cid = lax.axis_index("c")
    sid = lax.axis_index("s")

    def zfill(i, carry):
      zbuf[i, pl.ds(0, 16)] = jnp.zeros((16,), jnp.float32)
      zbuf[i, pl.ds(16, 16)] = jnp.zeros((16,), jnp.float32)
      return carry

    lax.fori_loop(0, ZROWS, zfill, 0)

    for step in range(half):
      for z in range(RPT_NODES // ZROWS):
        pltpu.sync_copy(zbuf, agg_s.at[pl.ds(sid * RPT_NODES + z * ZROWS, ZROWS), :])
      plsc.subcore_barrier()

      for sc_id in (0, 1):
        c = step if sc_id == 0 else half + step
        hc = h_hbm.at[c]

        @pl.when(cid == sc_id)
        def _():
          def body(r, carry):
            grow = sid * ROWS_PER_TILE + r * IDX_BLK
            pltpu.sync_copy(src_hbm.at[pl.ds(grow, IDX_BLK), :], srcb)
            pltpu.sync_copy(dst_hbm.at[pl.ds(grow, IDX_BLK), :], dstb)
            for j in range(IDX_BLK):
              pltpu.async_copy(hc.at[srcb.at[j]], rows, gsem).wait()
              pltpu.sync_copy(rows, agg_s.at[dstb.at[j]], add=True)
            return carry

          lax.fori_loop(0, N_OUTER, body, 0)

      plsc.subcore_barrier()

      for sc_id in (0, 1):
        c = step if sc_id == 0 else half + step

        @pl.when(cid == sc_id)
        def _():
          for z in range(RPT_NODES // ZROWS):
            sl = pl.ds(sid * RPT_NODES + z * ZROWS, ZROWS)
            pltpu.sync_copy(agg_s.at[sl, :], out_hbm.at[c].at[sl, :])

      plsc.subcore_barrier()

  return k


_sc_agg2 = _make_sc_agg(2)
_sc_agg8 = _make_sc_agg(8)

ROW_BLK = 2000
N_BLKS = N // ROW_BLK


def _conv_body(x_ref, agg_ref, invdeg_ref, ws_ref, wn_ref, b_ref, h_ref, st_ref):
  hn = agg_ref[...] * invdeg_ref[...]
  h = (jnp.dot(x_ref[...], ws_ref[...], preferred_element_type=jnp.float32)
       + jnp.dot(hn, wn_ref[...], preferred_element_type=jnp.float32)
       + b_ref[...])
  h_ref[...] = h

  @pl.when(pl.program_id(0) == 0)
  def _():
    st_ref[...] = jnp.zeros_like(st_ref)

  st_ref[0:1, :] = st_ref[0:1, :] + jnp.sum(h, axis=0, keepdims=True)
  st_ref[1:2, :] = st_ref[1:2, :] + jnp.sum(h * h, axis=0, keepdims=True)


def _conv_call(x, agg, invdeg, ws, wn, b):
  kin = x.shape[1]
  return pl.pallas_call(
      _conv_body,
      grid=(N_BLKS,),
      in_specs=[
          pl.BlockSpec((ROW_BLK, kin), lambda i: (i, 0)),
          pl.BlockSpec((ROW_BLK, HID if kin == HID else kin), lambda i: (i, 0)),
          pl.BlockSpec((ROW_BLK, 1), lambda i: (i, 0)),
          pl.BlockSpec((kin, HID), lambda i: (0, 0)),
          pl.BlockSpec((HID if kin == HID else kin, HID), lambda i: (0, 0)),
          pl.BlockSpec((1, HID), lambda i: (0, 0)),
      ],
      out_specs=[
          pl.BlockSpec((ROW_BLK, HID), lambda i: (i, 0)),
          pl.BlockSpec((8, HID), lambda i: (0, 0)),
      ],
      out_shape=[
          jax.ShapeDtypeStruct((N, HID), jnp.float32),
          jax.ShapeDtypeStruct((8, HID), jnp.float32),
      ],
      compiler_params=pltpu.CompilerParams(
          dimension_semantics=("arbitrary",)),
  )(x, agg, invdeg, ws, wn, b.reshape(1, HID))


def _bn_body(st_ref, g_ref, bt_ref, h_ref, out_ref):
  mu = st_ref[0:1, :] * (1.0 / N)
  var = st_ref[1:2, :] * (1.0 / N) - mu * mu
  scale = g_ref[...] * lax.rsqrt(var + EPS)
  y = (h_ref[...] - mu) * scale + bt_ref[...]
  out_ref[...] = jnp.where(y >= 0, y, SLOPE * y)


def _bn_call(h, st, gamma, beta):
  return pl.pallas_call(
      _bn_body,
      grid=(N_BLKS,),
      in_specs=[
          pl.BlockSpec((8, HID), lambda i: (0, 0)),
          pl.BlockSpec((1, HID), lambda i: (0, 0)),
          pl.BlockSpec((1, HID), lambda i: (0, 0)),
          pl.BlockSpec((ROW_BLK, HID), lambda i: (i, 0)),
      ],
      out_specs=pl.BlockSpec((ROW_BLK, HID), lambda i: (i, 0)),
      out_shape=jax.ShapeDtypeStruct((N, HID), jnp.float32),
      compiler_params=pltpu.CompilerParams(
          dimension_semantics=("arbitrary",)),
  )(st, gamma.reshape(1, HID), beta.reshape(1, HID), h)


def _bn_sum_body(st_ref, g_ref, bt_ref, h_ref, cs_ref):
  mu = st_ref[0:1, :] * (1.0 / N)
  var = st_ref[1:2, :] * (1.0 / N) - mu * mu
  scale = g_ref[...] * lax.rsqrt(var + EPS)
  y = (h_ref[...] - mu) * scale + bt_ref[...]
  y = jnp.where(y >= 0, y, SLOPE * y)

  @pl.when(pl.program_id(0) == 0)
  def _():
    cs_ref[...] = jnp.zeros_like(cs_ref)

  cs_ref[0:1, :] = cs_ref[0:1, :] + jnp.sum(y, axis=0, keepdims=True)


def _bn_sum_call(h, st, gamma, beta):
  return pl.pallas_call(
      _bn_sum_body,
      grid=(N_BLKS,),
      in_specs=[
          pl.BlockSpec((8, HID), lambda i: (0, 0)),
          pl.BlockSpec((1, HID), lambda i: (0, 0)),
          pl.BlockSpec((1, HID), lambda i: (0, 0)),
          pl.BlockSpec((ROW_BLK, HID), lambda i: (i, 0)),
      ],
      out_specs=pl.BlockSpec((8, HID), lambda i: (0, 0)),
      out_shape=jax.ShapeDtypeStruct((8, HID), jnp.float32),
      compiler_params=pltpu.CompilerParams(
          dimension_semantics=("arbitrary",)),
  )(st, gamma.reshape(1, HID), beta.reshape(1, HID), h)


def _head_body(cs_ref, w1_ref, b1_ref, w2_ref, b2_ref, w3_ref, b3_ref, out_ref):
  hg = cs_ref[0:1, :] * (1.0 / N)
  y = jnp.dot(hg, w1_ref[...], preferred_element_type=jnp.float32) + b1_ref[...]
  y = jnp.where(y >= 0, y, SLOPE * y)
  y = jnp.dot(y, w2_ref[...], preferred_element_type=jnp.float32) + b2_ref[...]
  y = jnp.where(y >= 0, y, SLOPE * y)
  y = jnp.dot(y, w3_ref[...], preferred_element_type=jnp.float32) + b3_ref[...]
  out_ref[...] = y


def _head_call(cs, w1, b1, w2, b2, w3p, b3p):
  return pl.pallas_call(
      _head_body,
      out_shape=jax.ShapeDtypeStruct((1, 128), jnp.float32),
  )(cs, w1, b1.reshape(1, HID), w2, b2.reshape(1, 1024), w3p, b3p)


def _chunked(h):
  f = h.shape[1]
  hp = jnp.concatenate([h, jnp.zeros((NPAD - N, f), jnp.float32)], axis=0)
  return hp.reshape(NPAD, f // 32, 32).transpose(1, 0, 2)


def _flat(agg):
  c = agg.shape[0]
  return agg[:, :N].transpose(1, 0, 2).reshape(N, 32 * c)


def kernel(x, edge_index, Ws1, Wn1, bc1, Ws2, Wn2, bc2, Ws3, Wn3, bc3,
           gamma1, beta1, gamma2, beta2, gamma3, beta3,
           fc1_W, fc1_b, fc2_W, fc2_b, fc3_W, fc3_b):
  src = edge_index[0]
  dst = edge_index[1]
  npad_e = EROWS * 128 - E
  src2d = jnp.concatenate([src, jnp.full((npad_e,), N, jnp.int32)]).reshape(EROWS, 128)
  dst2d = jnp.concatenate([dst, jnp.zeros((npad_e,), jnp.int32)]).reshape(EROWS, 128)

  x_pad = jnp.concatenate([x, jnp.ones((N, 1), jnp.float32)], axis=1)
  ws1p = jnp.zeros((64, HID), jnp.float32).at[:63].set(Ws1)
  wn1p = jnp.zeros((64, HID), jnp.float32).at[:63].set(Wn1)

  agg1 = _flat(_sc_agg2(_chunked(x_pad), src2d, dst2d))
  deg = agg1[:, 63]
  invdeg = (1.0 / jnp.maximum(deg, 1.0)).reshape(N, 1)

  h1, st1 = _conv_call(x_pad, agg1, invdeg, ws1p, wn1p, bc1)
  h1 = _bn_call(h1, st1, gamma1, beta1)

  agg2 = _flat(_sc_agg8(_chunked(h1), src2d, dst2d))
  h2, st2 = _conv_call(h1, agg2, invdeg, Ws2, Wn2, bc2)
  h2 = _bn_call(h2, st2, gamma2, beta2)

  agg3 = _flat(_sc_agg8(_chunked(h2), src2d, dst2d))
  h3, st3 = _conv_call(h2, agg3, invdeg, Ws3, Wn3, bc3)
  cs3 = _bn_sum_call(h3, st3, gamma3, beta3)

  w3p = jnp.zeros((1024, 128), jnp.float32).at[:, :N_CLASSES].set(fc3_W)
  b3p = jnp.zeros((1, 128), jnp.float32).at[0, :N_CLASSES].set(fc3_b)
  y = _head_call(cs3, fc1_W, fc1_b, fc2_W, fc2_b, w3p, b3p)
  return y[:, :N_CLASSES]


# speedup vs baseline: 3.6308x; 1.3591x over previous
import functools

import jax
import jax.numpy as jnp
from jax import lax
from jax.experimental import pallas as pl
from jax.experimental.pallas import tpu as pltpu
from jax.experimental.pallas import tpu_sc as plsc

N = 50000
E = 800000
HID = 256
N_CLASSES = 18
EPS = 1e-5
SLOPE = 0.01

EROWS = 6272
ROWS_PER_TILE = EROWS // 16
G = 3
N_OUTER = 65
N_TAIL = 2
NPAD = N + 8
NACC = 50048
RPT_NODES = NACC // 16
ZROWS = 184


def _make_sc_agg(n_chunks):
  half = n_chunks // 2
  mesh = plsc.VectorSubcoreMesh(core_axis_name="c", subcore_axis_name="s")

  @functools.partial(
      pl.kernel,
      mesh=mesh,
      out_type=jax.ShapeDtypeStruct((n_chunks, NACC, 32), jnp.float32),
      scratch_types=[
          pltpu.VMEM_SHARED((NACC, 32), jnp.float32),
          pltpu.VMEM((2 * G, 128), jnp.int32),
          pltpu.VMEM((2 * G, 128), jnp.int32),
          pltpu.VMEM((2 * G, 128, 32), jnp.float32),
          pltpu.SemaphoreType.DMA,
          pltpu.SemaphoreType.DMA,
      ],
      compiler_params=pltpu.CompilerParams(use_tc_tiling_on_sc=False),
  )
  def k(h_hbm, src_hbm, dst_hbm, out_hbm, agg_s, srcb, dstb, rows,
        gsem, ssem):
    cid = lax.axis_index("c")
    sid = lax.axis_index("s")

    for step in range(half):
      def zfill(i, carry):
        rows[0, i, pl.ds(0, 16)] = jnp.zeros((16,), jnp.float32)
        rows[0, i, pl.ds(16, 16)] = jnp.zeros((16,), jnp.float32)
        return carry

      lax.fori_loop(0, 128, zfill, 0)
      zbase = sid * RPT_NODES
      for z in range(RPT_NODES // 128):
        pltpu.async_copy(rows.at[0], agg_s.at[pl.ds(zbase + z * 128, 128), :],
                         gsem)
      for z in range(RPT_NODES // 128):
        pltpu.make_async_copy(rows.at[0],
                              agg_s.at[pl.ds(zbase + z * 128, 128), :],
                              gsem).wait()
      zrem = RPT_NODES % 128
      pltpu.sync_copy(rows.at[0].at[pl.ds(0, zrem), :],
                      agg_s.at[pl.ds(zbase + RPT_NODES - zrem, zrem), :])
      plsc.subcore_barrier()

      for sc_id in (0, 1):
        c = step if sc_id == 0 else half + step
        hc = h_hbm.at[c]

        @pl.when(cid == sc_id)
        def _():
          def stage(grow, half_i, n=G):
            sl = pl.ds(half_i * G, n)
            pltpu.sync_copy(src_hbm.at[pl.ds(grow, n), :], srcb.at[sl, :])
            pltpu.sync_copy(dst_hbm.at[pl.ds(grow, n), :], dstb.at[sl, :])

          def fire_gathers(half_i, n=G):
            for j in range(n):
              b = half_i * G + j
              pltpu.async_copy(hc.at[srcb.at[b]], rows.at[b], gsem)

          def drain_gathers(half_i, n=G):
            for j in range(n):
              b = half_i * G + j
              pltpu.make_async_copy(hc.at[srcb.at[b]], rows.at[b], gsem).wait()

          def fire_scatters(half_i, n=G):
            for j in range(n):
              b = half_i * G + j
              pltpu.async_copy(rows.at[b], agg_s.at[dstb.at[b]], ssem, add=True)

          def drain_scatters(half_i, n=G):
            for j in range(n):
              b = half_i * G + j
              pltpu.make_async_copy(rows.at[b], agg_s.at[dstb.at[b]], ssem,
                                    ).wait()

          def body(t, carry):
            grow = sid * ROWS_PER_TILE + t * (2 * G)
            stage(grow, 0)
            fire_gathers(0)
            stage(grow + G, 1)
            drain_gathers(0)
            fire_scatters(0)
            fire_gathers(1)
            drain_gathers(1)
            drain_scatters(0)
            fire_scatters(1)
            drain_scatters(1)
            return carry

          lax.fori_loop(0, N_OUTER, body, 0)
          tail = sid * ROWS_PER_TILE + N_OUTER * 2 * G
          stage(tail, 0, N_TAIL)
          fire_gathers(0, N_TAIL)
          drain_gathers(0, N_TAIL)
          fire_scatters(0, N_TAIL)
          drain_scatters(0, N_TAIL)

      plsc.subcore_barrier()

      for sc_id in (0, 1):
        c = step if sc_id == 0 else half + step

        @pl.when(cid == sc_id)
        def _():
          for z in range(RPT_NODES // ZROWS):
            sl = pl.ds(sid * RPT_NODES + z * ZROWS, ZROWS)
            pltpu.sync_copy(agg_s.at[sl, :], out_hbm.at[c].at[sl, :])

      plsc.subcore_barrier()

  return k


_sc_agg2 = _make_sc_agg(2)
_sc_agg8 = _make_sc_agg(8)

ROW_BLK = 2000
N_BLKS = N // ROW_BLK


def _conv_body(x_ref, agg_ref, invdeg_ref, ws_ref, wn_ref, b_ref, h_ref, st_ref):
  hn = agg_ref[...] * invdeg_ref[...]
  h = (jnp.dot(x_ref[...], ws_ref[...], preferred_element_type=jnp.float32)
       + jnp.dot(hn, wn_ref[...], preferred_element_type=jnp.float32)
       + b_ref[...])
  h_ref[...] = h

  @pl.when(pl.program_id(0) == 0)
  def _():
    st_ref[...] = jnp.zeros_like(st_ref)

  st_ref[0:1, :] = st_ref[0:1, :] + jnp.sum(h, axis=0, keepdims=True)
  st_ref[1:2, :] = st_ref[1:2, :] + jnp.sum(h * h, axis=0, keepdims=True)


def _conv_call(x, agg, invdeg, ws, wn, b):
  kin = x.shape[1]
  return pl.pallas_call(
      _conv_body,
      grid=(N_BLKS,),
      in_specs=[
          pl.BlockSpec((ROW_BLK, kin), lambda i: (i, 0)),
          pl.BlockSpec((ROW_BLK, HID if kin == HID else kin), lambda i: (i, 0)),
          pl.BlockSpec((ROW_BLK, 1), lambda i: (i, 0)),
          pl.BlockSpec((kin, HID), lambda i: (0, 0)),
          pl.BlockSpec((HID if kin == HID else kin, HID), lambda i: (0, 0)),
          pl.BlockSpec((1, HID), lambda i: (0, 0)),
      ],
      out_specs=[
          pl.BlockSpec((ROW_BLK, HID), lambda i: (i, 0)),
          pl.BlockSpec((8, HID), lambda i: (0, 0)),
      ],
      out_shape=[
          jax.ShapeDtypeStruct((N, HID), jnp.float32),
          jax.ShapeDtypeStruct((8, HID), jnp.float32),
      ],
      compiler_params=pltpu.CompilerParams(
          dimension_semantics=("arbitrary",)),
  )(x, agg, invdeg, ws, wn, b.reshape(1, HID))


def _bn_body(st_ref, g_ref, bt_ref, h_ref, out_ref):
  mu = st_ref[0:1, :] * (1.0 / N)
  var = st_ref[1:2, :] * (1.0 / N) - mu * mu
  scale = g_ref[...] * lax.rsqrt(var + EPS)
  y = (h_ref[...] - mu) * scale + bt_ref[...]
  out_ref[...] = jnp.where(y >= 0, y, SLOPE * y)


def _bn_call(h, st, gamma, beta):
  return pl.pallas_call(
      _bn_body,
      grid=(N_BLKS,),
      in_specs=[
          pl.BlockSpec((8, HID), lambda i: (0, 0)),
          pl.BlockSpec((1, HID), lambda i: (0, 0)),
          pl.BlockSpec((1, HID), lambda i: (0, 0)),
          pl.BlockSpec((ROW_BLK, HID), lambda i: (i, 0)),
      ],
      out_specs=pl.BlockSpec((ROW_BLK, HID), lambda i: (i, 0)),
      out_shape=jax.ShapeDtypeStruct((N, HID), jnp.float32),
      compiler_params=pltpu.CompilerParams(
          dimension_semantics=("arbitrary",)),
  )(st, gamma.reshape(1, HID), beta.reshape(1, HID), h)


def _bn_sum_body(st_ref, g_ref, bt_ref, h_ref, cs_ref):
  mu = st_ref[0:1, :] * (1.0 / N)
  var = st_ref[1:2, :] * (1.0 / N) - mu * mu
  scale = g_ref[...] * lax.rsqrt(var + EPS)
  y = (h_ref[...] - mu) * scale + bt_ref[...]
  y = jnp.where(y >= 0, y, SLOPE * y)

  @pl.when(pl.program_id(0) == 0)
  def _():
    cs_ref[...] = jnp.zeros_like(cs_ref)

  cs_ref[0:1, :] = cs_ref[0:1, :] + jnp.sum(y, axis=0, keepdims=True)


def _bn_sum_call(h, st, gamma, beta):
  return pl.pallas_call(
      _bn_sum_body,
      grid=(N_BLKS,),
      in_specs=[
          pl.BlockSpec((8, HID), lambda i: (0, 0)),
          pl.BlockSpec((1, HID), lambda i: (0, 0)),
          pl.BlockSpec((1, HID), lambda i: (0, 0)),
          pl.BlockSpec((ROW_BLK, HID), lambda i: (i, 0)),
      ],
      out_specs=pl.BlockSpec((8, HID), lambda i: (0, 0)),
      out_shape=jax.ShapeDtypeStruct((8, HID), jnp.float32),
      compiler_params=pltpu.CompilerParams(
          dimension_semantics=("arbitrary",)),
  )(st, gamma.reshape(1, HID), beta.reshape(1, HID), h)


def _head_body(cs_ref, w1_ref, b1_ref, w2_ref, b2_ref, w3_ref, b3_ref, out_ref):
  hg = cs_ref[0:1, :] * (1.0 / N)
  y = jnp.dot(hg, w1_ref[...], preferred_element_type=jnp.float32) + b1_ref[...]
  y = jnp.where(y >= 0, y, SLOPE * y)
  y = jnp.dot(y, w2_ref[...], preferred_element_type=jnp.float32) + b2_ref[...]
  y = jnp.where(y >= 0, y, SLOPE * y)
  y = jnp.dot(y, w3_ref[...], preferred_element_type=jnp.float32) + b3_ref[...]
  out_ref[...] = y


def _head_call(cs, w1, b1, w2, b2, w3p, b3p):
  return pl.pallas_call(
      _head_body,
      out_shape=jax.ShapeDtypeStruct((1, 128), jnp.float32),
  )(cs, w1, b1.reshape(1, HID), w2, b2.reshape(1, 1024), w3p, b3p)


def _chunked(h):
  f = h.shape[1]
  hp = jnp.concatenate([h, jnp.zeros((NPAD - N, f), jnp.float32)], axis=0)
  return hp.reshape(NPAD, f // 32, 32).transpose(1, 0, 2)


def _flat(agg):
  c = agg.shape[0]
  return agg[:, :N].transpose(1, 0, 2).reshape(N, 32 * c)


def kernel(x, edge_index, Ws1, Wn1, bc1, Ws2, Wn2, bc2, Ws3, Wn3, bc3,
           gamma1, beta1, gamma2, beta2, gamma3, beta3,
           fc1_W, fc1_b, fc2_W, fc2_b, fc3_W, fc3_b):
  src = edge_index[0]
  dst = edge_index[1]
  npad_e = EROWS * 128 - E
  src2d = jnp.concatenate([src, jnp.full((npad_e,), N, jnp.int32)]).reshape(EROWS, 128)
  dst2d = jnp.concatenate([dst, jnp.zeros((npad_e,), jnp.int32)]).reshape(EROWS, 128)

  x_pad = jnp.concatenate([x, jnp.ones((N, 1), jnp.float32)], axis=1)
  ws1p = jnp.zeros((64, HID), jnp.float32).at[:63].set(Ws1)
  wn1p = jnp.zeros((64, HID), jnp.float32).at[:63].set(Wn1)

  agg1 = _flat(_sc_agg2(_chunked(x_pad), src2d, dst2d))
  deg = agg1[:, 63]
  invdeg = (1.0 / jnp.maximum(deg, 1.0)).reshape(N, 1)

  h1, st1 = _conv_call(x_pad, agg1, invdeg, ws1p, wn1p, bc1)
  h1 = _bn_call(h1, st1, gamma1, beta1)

  agg2 = _flat(_sc_agg8(_chunked(h1), src2d, dst2d))
  h2, st2 = _conv_call(h1, agg2, invdeg, Ws2, Wn2, bc2)
  h2 = _bn_call(h2, st2, gamma2, beta2)

  agg3 = _flat(_sc_agg8(_chunked(h2), src2d, dst2d))
  h3, st3 = _conv_call(h2, agg3, invdeg, Ws3, Wn3, bc3)
  cs3 = _bn_sum_call(h3, st3, gamma3, beta3)

  w3p = jnp.zeros((1024, 128), jnp.float32).at[:, :N_CLASSES].set(fc3_W)
  b3p = jnp.zeros((1, 128), jnp.float32).at[0, :N_CLASSES].set(fc3_b)
  y = _head_call(cs3, fc1_W, fc1_b, fc2_W, fc2_b, w3p, b3p)
  return y[:, :N_CLASSES]


# speedup vs baseline: 4.2908x; 1.1818x over previous
import functools

import jax
import jax.numpy as jnp
from jax import lax
from jax.experimental import pallas as pl
from jax.experimental.pallas import tpu as pltpu
from jax.experimental.pallas import tpu_sc as plsc

N = 50000
E = 800000
HID = 256
N_CLASSES = 18
EPS = 1e-5
SLOPE = 0.01

EROWS = 6272
ROWS_PER_TILE = EROWS // 16
G = 3
N_OUTER = 65
N_TAIL = 2
NPAD = N + 8
NACC = 50048
RPT_NODES = NACC // 16
ZROWS = 184


def _make_sc_agg(n_chunks):
  half = n_chunks // 2
  mesh = plsc.VectorSubcoreMesh(core_axis_name="c", subcore_axis_name="s")

  @functools.partial(
      pl.kernel,
      mesh=mesh,
      out_type=jax.ShapeDtypeStruct((NACC, 32 * n_chunks), jnp.float32),
      scratch_types=[
          pltpu.VMEM_SHARED((NACC, 32), jnp.float32),
          pltpu.VMEM((2 * G, 128), jnp.int32),
          pltpu.VMEM((2 * G, 128), jnp.int32),
          pltpu.VMEM((2 * G, 128, 32), jnp.float32),
          pltpu.SemaphoreType.DMA,
          pltpu.SemaphoreType.DMA,
      ],
      compiler_params=pltpu.CompilerParams(use_tc_tiling_on_sc=False),
  )
  def k(h_hbm, src_hbm, dst_hbm, out_hbm, agg_s, srcb, dstb, rows,
        gsem, ssem):
    cid = lax.axis_index("c")
    sid = lax.axis_index("s")

    for step in range(half):
      def zfill(i, carry):
        rows[0, i, pl.ds(0, 16)] = jnp.zeros((16,), jnp.float32)
        rows[0, i, pl.ds(16, 16)] = jnp.zeros((16,), jnp.float32)
        return carry

      lax.fori_loop(0, 128, zfill, 0)
      zbase = sid * RPT_NODES
      for z in range(RPT_NODES // 128):
        pltpu.async_copy(rows.at[0], agg_s.at[pl.ds(zbase + z * 128, 128), :],
                         gsem)
      for z in range(RPT_NODES // 128):
        pltpu.make_async_copy(rows.at[0],
                              agg_s.at[pl.ds(zbase + z * 128, 128), :],
                              gsem).wait()
      zrem = RPT_NODES % 128
      pltpu.sync_copy(rows.at[0].at[pl.ds(0, zrem), :],
                      agg_s.at[pl.ds(zbase + RPT_NODES - zrem, zrem), :])
      plsc.subcore_barrier()

      for sc_id in (0, 1):
        c = step if sc_id == 0 else half + step
        hc = h_hbm.at[c]

        @pl.when(cid == sc_id)
        def _():
          def stage(grow, half_i, n=G):
            sl = pl.ds(half_i * G, n)
            pltpu.sync_copy(src_hbm.at[pl.ds(grow, n), :], srcb.at[sl, :])
            pltpu.sync_copy(dst_hbm.at[pl.ds(grow, n), :], dstb.at[sl, :])

          def fire_gathers(half_i, n=G):
            for j in range(n):
              b = half_i * G + j
              pltpu.async_copy(hc.at[srcb.at[b]], rows.at[b], gsem)

          def drain_gathers(half_i, n=G):
            for j in range(n):
              b = half_i * G + j
              pltpu.make_async_copy(hc.at[srcb.at[b]], rows.at[b], gsem).wait()

          def fire_scatters(half_i, n=G):
            for j in range(n):
              b = half_i * G + j
              pltpu.async_copy(rows.at[b], agg_s.at[dstb.at[b]], ssem, add=True)

          def drain_scatters(half_i, n=G):
            for j in range(n):
              b = half_i * G + j
              pltpu.make_async_copy(rows.at[b], agg_s.at[dstb.at[b]], ssem,
                                    ).wait()

          def body(t, carry):
            grow = sid * ROWS_PER_TILE + t * (2 * G)
            stage(grow, 0)
            fire_gathers(0)
            stage(grow + G, 1)
            drain_gathers(0)
            fire_scatters(0)
            fire_gathers(1)
            drain_gathers(1)
            drain_scatters(0)
            fire_scatters(1)
            drain_scatters(1)
            return carry

          lax.fori_loop(0, N_OUTER, body, 0)
          tail = sid * ROWS_PER_TILE + N_OUTER * 2 * G
          stage(tail, 0, N_TAIL)
          fire_gathers(0, N_TAIL)
          drain_gathers(0, N_TAIL)
          fire_scatters(0, N_TAIL)
          drain_scatters(0, N_TAIL)

      plsc.subcore_barrier()

      for sc_id in (0, 1):
        c = step if sc_id == 0 else half + step

        @pl.when(cid == sc_id)
        def _():
          for z in range(RPT_NODES // ZROWS):
            sl = pl.ds(sid * RPT_NODES + z * ZROWS, ZROWS)
            pltpu.sync_copy(agg_s.at[sl, :], out_hbm.at[sl, pl.ds(32 * c, 32)])

      plsc.subcore_barrier()

  return k


_sc_agg2 = _make_sc_agg(2)
_sc_agg8 = _make_sc_agg(8)

ROW_BLK = 2000
N_BLKS = N // ROW_BLK


def _conv_body(x_ref, agg_ref, invdeg_ref, ws_ref, wn_ref, b_ref, h_ref, st_ref):
  hn = agg_ref[...] * invdeg_ref[...]
  h = (jnp.dot(x_ref[...], ws_ref[...], preferred_element_type=jnp.float32)
       + jnp.dot(hn, wn_ref[...], preferred_element_type=jnp.float32)
       + b_ref[...])
  h_ref[...] = h

  @pl.when(pl.program_id(0) == 0)
  def _():
    st_ref[...] = jnp.zeros_like(st_ref)

  st_ref[0:1, :] = st_ref[0:1, :] + jnp.sum(h, axis=0, keepdims=True)
  st_ref[1:2, :] = st_ref[1:2, :] + jnp.sum(h * h, axis=0, keepdims=True)


def _conv_call(x, agg, invdeg, ws, wn, b):
  kin = x.shape[1]
  return pl.pallas_call(
      _conv_body,
      grid=(N_BLKS,),
      in_specs=[
          pl.BlockSpec((ROW_BLK, kin), lambda i: (i, 0)),
          pl.BlockSpec((ROW_BLK, HID if kin == HID else kin), lambda i: (i, 0)),
          pl.BlockSpec((ROW_BLK, 1), lambda i: (i, 0)),
          pl.BlockSpec((kin, HID), lambda i: (0, 0)),
          pl.BlockSpec((HID if kin == HID else kin, HID), lambda i: (0, 0)),
          pl.BlockSpec((1, HID), lambda i: (0, 0)),
      ],
      out_specs=[
          pl.BlockSpec((ROW_BLK, HID), lambda i: (i, 0)),
          pl.BlockSpec((8, HID), lambda i: (0, 0)),
      ],
      out_shape=[
          jax.ShapeDtypeStruct((N, HID), jnp.float32),
          jax.ShapeDtypeStruct((8, HID), jnp.float32),
      ],
      compiler_params=pltpu.CompilerParams(
          dimension_semantics=("arbitrary",)),
  )(x, agg, invdeg, ws, wn, b.reshape(1, HID))


def _bn_body(st_ref, g_ref, bt_ref, h_ref, out_ref):
  mu = st_ref[0:1, :] * (1.0 / N)
  var = st_ref[1:2, :] * (1.0 / N) - mu * mu
  scale = g_ref[...] * lax.rsqrt(var + EPS)
  y = (h_ref[...] - mu) * scale + bt_ref[...]
  out_ref[...] = jnp.where(y >= 0, y, SLOPE * y)


def _bn_call(h, st, gamma, beta):
  return pl.pallas_call(
      _bn_body,
      grid=(N_BLKS,),
      in_specs=[
          pl.BlockSpec((8, HID), lambda i: (0, 0)),
          pl.BlockSpec((1, HID), lambda i: (0, 0)),
          pl.BlockSpec((1, HID), lambda i: (0, 0)),
          pl.BlockSpec((ROW_BLK, HID), lambda i: (i, 0)),
      ],
      out_specs=pl.BlockSpec((ROW_BLK, HID), lambda i: (i, 0)),
      out_shape=jax.ShapeDtypeStruct((N, HID), jnp.float32),
      compiler_params=pltpu.CompilerParams(
          dimension_semantics=("arbitrary",)),
  )(st, gamma.reshape(1, HID), beta.reshape(1, HID), h)


def _bn_sum_body(st_ref, g_ref, bt_ref, h_ref, cs_ref):
  mu = st_ref[0:1, :] * (1.0 / N)
  var = st_ref[1:2, :] * (1.0 / N) - mu * mu
  scale = g_ref[...] * lax.rsqrt(var + EPS)
  y = (h_ref[...] - mu) * scale + bt_ref[...]
  y = jnp.where(y >= 0, y, SLOPE * y)

  @pl.when(pl.program_id(0) == 0)
  def _():
    cs_ref[...] = jnp.zeros_like(cs_ref)

  cs_ref[0:1, :] = cs_ref[0:1, :] + jnp.sum(y, axis=0, keepdims=True)


def _bn_sum_call(h, st, gamma, beta):
  return pl.pallas_call(
      _bn_sum_body,
      grid=(N_BLKS,),
      in_specs=[
          pl.BlockSpec((8, HID), lambda i: (0, 0)),
          pl.BlockSpec((1, HID), lambda i: (0, 0)),
          pl.BlockSpec((1, HID), lambda i: (0, 0)),
          pl.BlockSpec((ROW_BLK, HID), lambda i: (i, 0)),
      ],
      out_specs=pl.BlockSpec((8, HID), lambda i: (0, 0)),
      out_shape=jax.ShapeDtypeStruct((8, HID), jnp.float32),
      compiler_params=pltpu.CompilerParams(
          dimension_semantics=("arbitrary",)),
  )(st, gamma.reshape(1, HID), beta.reshape(1, HID), h)


def _head_body(cs_ref, w1_ref, b1_ref, w2_ref, b2_ref, w3_ref, b3_ref, out_ref):
  hg = cs_ref[0:1, :] * (1.0 / N)
  y = jnp.dot(hg, w1_ref[...], preferred_element_type=jnp.float32) + b1_ref[...]
  y = jnp.where(y >= 0, y, SLOPE * y)
  y = jnp.dot(y, w2_ref[...], preferred_element_type=jnp.float32) + b2_ref[...]
  y = jnp.where(y >= 0, y, SLOPE * y)
  y = jnp.dot(y, w3_ref[...], preferred_element_type=jnp.float32) + b3_ref[...]
  out_ref[...] = y


def _head_call(cs, w1, b1, w2, b2, w3p, b3p):
  return pl.pallas_call(
      _head_body,
      out_shape=jax.ShapeDtypeStruct((1, 128), jnp.float32),
  )(cs, w1, b1.reshape(1, HID), w2, b2.reshape(1, 1024), w3p, b3p)


def _chunked(h):
  f = h.shape[1]
  return h.reshape(N, f // 32, 32).transpose(1, 0, 2)


def kernel(x, edge_index, Ws1, Wn1, bc1, Ws2, Wn2, bc2, Ws3, Wn3, bc3,
           gamma1, beta1, gamma2, beta2, gamma3, beta3,
           fc1_W, fc1_b, fc2_W, fc2_b, fc3_W, fc3_b):
  src = edge_index[0]
  dst = edge_index[1]
  npad_e = EROWS * 128 - E
  src2d = jnp.concatenate([src, jnp.zeros((npad_e,), jnp.int32)]).reshape(EROWS, 128)
  dst2d = jnp.concatenate([dst, jnp.full((npad_e,), N, jnp.int32)]).reshape(EROWS, 128)

  x_pad = jnp.concatenate([x, jnp.ones((N, 1), jnp.float32)], axis=1)
  ws1p = jnp.zeros((64, HID), jnp.float32).at[:63].set(Ws1)
  wn1p = jnp.zeros((64, HID), jnp.float32).at[:63].set(Wn1)

  agg1 = _sc_agg2(_chunked(x_pad), src2d, dst2d)[:N]
  deg = agg1[:, 63]
  invdeg = (1.0 / jnp.maximum(deg, 1.0)).reshape(N, 1)

  h1, st1 = _conv_call(x_pad, agg1, invdeg, ws1p, wn1p, bc1)
  h1 = _bn_call(h1, st1, gamma1, beta1)

  agg2 = _sc_agg8(_chunked(h1), src2d, dst2d)[:N]
  h2, st2 = _conv_call(h1, agg2, invdeg, Ws2, Wn2, bc2)
  h2 = _bn_call(h2, st2, gamma2, beta2)

  agg3 = _sc_agg8(_chunked(h2), src2d, dst2d)[:N]
  h3, st3 = _conv_call(h2, agg3, invdeg, Ws3, Wn3, bc3)
  cs3 = _bn_sum_call(h3, st3, gamma3, beta3)

  w3p = jnp.zeros((1024, 128), jnp.float32).at[:, :N_CLASSES].set(fc3_W)
  b3p = jnp.zeros((1, 128), jnp.float32).at[0, :N_CLASSES].set(fc3_b)
  y = _head_call(cs3, fc1_W, fc1_b, fc2_W, fc2_b, w3p, b3p)
  return y[:, :N_CLASSES]


# speedup vs baseline: 4.3842x; 1.0218x over previous
import functools

import jax
import jax.numpy as jnp
from jax import lax
from jax.experimental import pallas as pl
from jax.experimental.pallas import tpu as pltpu
from jax.experimental.pallas import tpu_sc as plsc

N = 50000
E = 800000
HID = 256
N_CLASSES = 18
EPS = 1e-5
SLOPE = 0.01

EROWS = 6272
ROWS_PER_TILE = EROWS // 16
G = 3
N_OUTER = 65
N_TAIL = 2
NPAD = N + 8
NACC = 50048
RPT_NODES = NACC // 16
ZROWS = 184


def _make_sc_agg(n_chunks):
  half = n_chunks // 2
  mesh = plsc.VectorSubcoreMesh(core_axis_name="c", subcore_axis_name="s")

  @functools.partial(
      pl.kernel,
      mesh=mesh,
      out_type=jax.ShapeDtypeStruct((NACC, 32 * n_chunks), jnp.float32),
      scratch_types=[
          pltpu.VMEM_SHARED((NACC, 32), jnp.float32),
          pltpu.VMEM((2 * G, 128), jnp.int32),
          pltpu.VMEM((2 * G, 128), jnp.int32),
          pltpu.VMEM((2 * G, 128, 32), jnp.float32),
          pltpu.SemaphoreType.DMA,
          pltpu.SemaphoreType.DMA,
      ],
      compiler_params=pltpu.CompilerParams(use_tc_tiling_on_sc=False),
  )
  def k(h_hbm, src_hbm, dst_hbm, out_hbm, agg_s, srcb, dstb, rows,
        gsem, ssem):
    cid = lax.axis_index("c")
    sid = lax.axis_index("s")

    for step in range(half):
      def zfill(i, carry):
        rows[0, i, pl.ds(0, 16)] = jnp.zeros((16,), jnp.float32)
        rows[0, i, pl.ds(16, 16)] = jnp.zeros((16,), jnp.float32)
        return carry

      lax.fori_loop(0, 128, zfill, 0)
      zbase = sid * RPT_NODES
      for z in range(RPT_NODES // 128):
        pltpu.async_copy(rows.at[0], agg_s.at[pl.ds(zbase + z * 128, 128), :],
                         gsem)
      for z in range(RPT_NODES // 128):
        pltpu.make_async_copy(rows.at[0],
                              agg_s.at[pl.ds(zbase + z * 128, 128), :],
                              gsem).wait()
      zrem = RPT_NODES % 128
      pltpu.sync_copy(rows.at[0].at[pl.ds(0, zrem), :],
                      agg_s.at[pl.ds(zbase + RPT_NODES - zrem, zrem), :])
      plsc.subcore_barrier()

      for sc_id in (0, 1):
        c = step if sc_id == 0 else half + step
        hc = h_hbm.at[c]

        @pl.when(cid == sc_id)
        def _():
          def stage(grow, half_i, n=G):
            sl = pl.ds(half_i * G, n)
            pltpu.sync_copy(src_hbm.at[pl.ds(grow, n), :], srcb.at[sl, :])
            pltpu.sync_copy(dst_hbm.at[pl.ds(grow, n), :], dstb.at[sl, :])

          def fire_gathers(half_i, n=G):
            for j in range(n):
              b = half_i * G + j
              pltpu.async_copy(hc.at[srcb.at[b]], rows.at[b], gsem)

          def drain_gathers(half_i, n=G):
            for j in range(n):
              b = half_i * G + j
              pltpu.make_async_copy(hc.at[srcb.at[b]], rows.at[b], gsem).wait()

          def fire_scatters(half_i, n=G):
            for j in range(n):
              b = half_i * G + j
              pltpu.async_copy(rows.at[b], agg_s.at[dstb.at[b]], ssem, add=True)

          def drain_scatters(half_i, n=G):
            for j in range(n):
              b = half_i * G + j
              pltpu.make_async_copy(rows.at[b], agg_s.at[dstb.at[b]], ssem,
                                    ).wait()

          def body(t, carry):
            grow = sid * ROWS_PER_TILE + t * (2 * G)
            stage(grow, 0)
            fire_gathers(0)
            stage(grow + G, 1)
            fire_gathers(1)
            drain_gathers(0)
            fire_scatters(0)
            drain_gathers(1)
            fire_scatters(1)
            drain_scatters(0)
            drain_scatters(1)
            return carry

          lax.fori_loop(0, N_OUTER, body, 0)
          tail = sid * ROWS_PER_TILE + N_OUTER * 2 * G
          stage(tail, 0, N_TAIL)
          fire_gathers(0, N_TAIL)
          drain_gathers(0, N_TAIL)
          fire_scatters(0, N_TAIL)
          drain_scatters(0, N_TAIL)

      plsc.subcore_barrier()

      for sc_id in (0, 1):
        c = step if sc_id == 0 else half + step

        @pl.when(cid == sc_id)
        def _():
          for z in range(RPT_NODES // ZROWS):
            sl = pl.ds(sid * RPT_NODES + z * ZROWS, ZROWS)
            pltpu.async_copy(agg_s.at[sl, :], out_hbm.at[sl, pl.ds(32 * c, 32)],
                             gsem)
          for z in range(RPT_NODES // ZROWS):
            sl = pl.ds(sid * RPT_NODES + z * ZROWS, ZROWS)
            pltpu.make_async_copy(agg_s.at[sl, :],
                                  out_hbm.at[sl, pl.ds(32 * c, 32)],
                                  gsem).wait()

      plsc.subcore_barrier()

  return k


_sc_agg2 = _make_sc_agg(2)
_sc_agg8 = _make_sc_agg(8)

ROW_BLK = 2000
N_BLKS = N // ROW_BLK


def _conv_body(x_ref, agg_ref, invdeg_ref, ws_ref, wn_ref, b_ref, h_ref, st_ref):
  hn = agg_ref[...] * invdeg_ref[...]
  h = (jnp.dot(x_ref[...], ws_ref[...], preferred_element_type=jnp.float32)
       + jnp.dot(hn, wn_ref[...], preferred_element_type=jnp.float32)
       + b_ref[...])
  h_ref[...] = h

  @pl.when(pl.program_id(0) == 0)
  def _():
    st_ref[...] = jnp.zeros_like(st_ref)

  st_ref[0:1, :] = st_ref[0:1, :] + jnp.sum(h, axis=0, keepdims=True)
  st_ref[1:2, :] = st_ref[1:2, :] + jnp.sum(h * h, axis=0, keepdims=True)


def _conv_call(x, agg, invdeg, ws, wn, b):
  kin = x.shape[1]
  return pl.pallas_call(
      _conv_body,
      grid=(N_BLKS,),
      in_specs=[
          pl.BlockSpec((ROW_BLK, kin), lambda i: (i, 0)),
          pl.BlockSpec((ROW_BLK, HID if kin == HID else kin), lambda i: (i, 0)),
          pl.BlockSpec((ROW_BLK, 1), lambda i: (i, 0)),
          pl.BlockSpec((kin, HID), lambda i: (0, 0)),
          pl.BlockSpec((HID if kin == HID else kin, HID), lambda i: (0, 0)),
          pl.BlockSpec((1, HID), lambda i: (0, 0)),
      ],
      out_specs=[
          pl.BlockSpec((ROW_BLK, HID), lambda i: (i, 0)),
          pl.BlockSpec((8, HID), lambda i: (0, 0)),
      ],
      out_shape=[
          jax.ShapeDtypeStruct((N, HID), jnp.float32),
          jax.ShapeDtypeStruct((8, HID), jnp.float32),
      ],
      compiler_params=pltpu.CompilerParams(
          dimension_semantics=("arbitrary",)),
  )(x, agg, invdeg, ws, wn, b.reshape(1, HID))


def _bn_body(st_ref, g_ref, bt_ref, h_ref, out_ref):
  mu = st_ref[0:1, :] * (1.0 / N)
  var = st_ref[1:2, :] * (1.0 / N) - mu * mu
  scale = g_ref[...] * lax.rsqrt(var + EPS)
  y = (h_ref[...] - mu) * scale + bt_ref[...]
  out_ref[...] = jnp.where(y >= 0, y, SLOPE * y)


def _bn_call(h, st, gamma, beta):
  return pl.pallas_call(
      _bn_body,
      grid=(N_BLKS,),
      in_specs=[
          pl.BlockSpec((8, HID), lambda i: (0, 0)),
          pl.BlockSpec((1, HID), lambda i: (0, 0)),
          pl.BlockSpec((1, HID), lambda i: (0, 0)),
          pl.BlockSpec((ROW_BLK, HID), lambda i: (i, 0)),
      ],
      out_specs=pl.BlockSpec((ROW_BLK, HID), lambda i: (i, 0)),
      out_shape=jax.ShapeDtypeStruct((N, HID), jnp.float32),
      compiler_params=pltpu.CompilerParams(
          dimension_semantics=("arbitrary",)),
  )(st, gamma.reshape(1, HID), beta.reshape(1, HID), h)


def _bn_sum_body(st_ref, g_ref, bt_ref, h_ref, cs_ref):
  mu = st_ref[0:1, :] * (1.0 / N)
  var = st_ref[1:2, :] * (1.0 / N) - mu * mu
  scale = g_ref[...] * lax.rsqrt(var + EPS)
  y = (h_ref[...] - mu) * scale + bt_ref[...]
  y = jnp.where(y >= 0, y, SLOPE * y)

  @pl.when(pl.program_id(0) == 0)
  def _():
    cs_ref[...] = jnp.zeros_like(cs_ref)

  cs_ref[0:1, :] = cs_ref[0:1, :] + jnp.sum(y, axis=0, keepdims=True)


def _bn_sum_call(h, st, gamma, beta):
  return pl.pallas_call(
      _bn_sum_body,
      grid=(N_BLKS,),
      in_specs=[
          pl.BlockSpec((8, HID), lambda i: (0, 0)),
          pl.BlockSpec((1, HID), lambda i: (0, 0)),
          pl.BlockSpec((1, HID), lambda i: (0, 0)),
          pl.BlockSpec((ROW_BLK, HID), lambda i: (i, 0)),
      ],
      out_specs=pl.BlockSpec((8, HID), lambda i: (0, 0)),
      out_shape=jax.ShapeDtypeStruct((8, HID), jnp.float32),
      compiler_params=pltpu.CompilerParams(
          dimension_semantics=("arbitrary",)),
  )(st, gamma.reshape(1, HID), beta.reshape(1, HID), h)


def _head_body(cs_ref, w1_ref, b1_ref, w2_ref, b2_ref, w3_ref, b3_ref, out_ref):
  hg = cs_ref[0:1, :] * (1.0 / N)
  y = jnp.dot(hg, w1_ref[...], preferred_element_type=jnp.float32) + b1_ref[...]
  y = jnp.where(y >= 0, y, SLOPE * y)
  y = jnp.dot(y, w2_ref[...], preferred_element_type=jnp.float32) + b2_ref[...]
  y = jnp.where(y >= 0, y, SLOPE * y)
  y = jnp.dot(y, w3_ref[...], preferred_element_type=jnp.float32) + b3_ref[...]
  out_ref[...] = y


def _head_call(cs, w1, b1, w2, b2, w3p, b3p):
  return pl.pallas_call(
      _head_body,
      out_shape=jax.ShapeDtypeStruct((1, 128), jnp.float32),
  )(cs, w1, b1.reshape(1, HID), w2, b2.reshape(1, 1024), w3p, b3p)


def _chunked(h):
  f = h.shape[1]
  return h.reshape(N, f // 32, 32).transpose(1, 0, 2)


def kernel(x, edge_index, Ws1, Wn1, bc1, Ws2, Wn2, bc2, Ws3, Wn3, bc3,
           gamma1, beta1, gamma2, beta2, gamma3, beta3,
           fc1_W, fc1_b, fc2_W, fc2_b, fc3_W, fc3_b):
  src = edge_index[0]
  dst = edge_index[1]
  npad_e = EROWS * 128 - E
  src2d = jnp.concatenate([src, jnp.zeros((npad_e,), jnp.int32)]).reshape(EROWS, 128)
  dst2d = jnp.concatenate([dst, jnp.full((npad_e,), N, jnp.int32)]).reshape(EROWS, 128)

  x_pad = jnp.concatenate([x, jnp.ones((N, 1), jnp.float32)], axis=1)
  ws1p = jnp.zeros((64, HID), jnp.float32).at[:63].set(Ws1)
  wn1p = jnp.zeros((64, HID), jnp.float32).at[:63].set(Wn1)

  agg1 = _sc_agg2(_chunked(x_pad), src2d, dst2d)[:N]
  deg = agg1[:, 63]
  invdeg = (1.0 / jnp.maximum(deg, 1.0)).reshape(N, 1)

  h1, st1 = _conv_call(x_pad, agg1, invdeg, ws1p, wn1p, bc1)
  h1 = _bn_call(h1, st1, gamma1, beta1)

  agg2 = _sc_agg8(_chunked(h1), src2d, dst2d)[:N]
  h2, st2 = _conv_call(h1, agg2, invdeg, Ws2, Wn2, bc2)
  h2 = _bn_call(h2, st2, gamma2, beta2)

  agg3 = _sc_agg8(_chunked(h2), src2d, dst2d)[:N]
  h3, st3 = _conv_call(h2, agg3, invdeg, Ws3, Wn3, bc3)
  cs3 = _bn_sum_call(h3, st3, gamma3, beta3)

  w3p = jnp.zeros((1024, 128), jnp.float32).at[:, :N_CLASSES].set(fc3_W)
  b3p = jnp.zeros((1, 128), jnp.float32).at[0, :N_CLASSES].set(fc3_b)
  y = _head_call(cs3, fc1_W, fc1_b, fc2_W, fc2_b, w3p, b3p)
  return y[:, :N_CLASSES]


# speedup vs baseline: 4.5394x; 1.0354x over previous
import functools

import jax
import jax.numpy as jnp
from jax import lax
from jax.experimental import pallas as pl
from jax.experimental.pallas import tpu as pltpu
from jax.experimental.pallas import tpu_sc as plsc

N = 50000
E = 800000
HID = 256
N_CLASSES = 18
EPS = 1e-5
SLOPE = 0.01

EROWS = 6272
ROWS_PER_TILE = EROWS // 16
G = 3
N_OUTER = 65
N_TAIL = 2
NPAD = N + 8
NACC = 50048
RPT_NODES = NACC // 16
ZROWS = 184


def _make_sc_agg(n_chunks):
  half = n_chunks // 2
  mesh = plsc.VectorSubcoreMesh(core_axis_name="c", subcore_axis_name="s")

  @functools.partial(
      pl.kernel,
      mesh=mesh,
      out_type=jax.ShapeDtypeStruct((NACC, 32 * n_chunks), jnp.float32),
      scratch_types=[
          pltpu.VMEM_SHARED((NACC, 32), jnp.float32),
          pltpu.VMEM((2 * G, 128), jnp.int32),
          pltpu.VMEM((2 * G, 128), jnp.int32),
          pltpu.VMEM((2 * G, 128, 32), jnp.float32),
          pltpu.SemaphoreType.DMA,
          pltpu.SemaphoreType.DMA,
      ],
      compiler_params=pltpu.CompilerParams(use_tc_tiling_on_sc=False),
  )
  def k(h_hbm, src_hbm, dst_hbm, out_hbm, agg_s, srcb, dstb, rows,
        gsem, ssem):
    cid = lax.axis_index("c")
    sid = lax.axis_index("s")

    for step in range(half):
      def zfill(i, carry):
        rows[0, i, pl.ds(0, 16)] = jnp.zeros((16,), jnp.float32)
        rows[0, i, pl.ds(16, 16)] = jnp.zeros((16,), jnp.float32)
        return carry

      lax.fori_loop(0, 128, zfill, 0)
      zbase = sid * RPT_NODES
      for z in range(RPT_NODES // 128):
        pltpu.async_copy(rows.at[0], agg_s.at[pl.ds(zbase + z * 128, 128), :],
                         gsem)
      for z in range(RPT_NODES // 128):
        pltpu.make_async_copy(rows.at[0],
                              agg_s.at[pl.ds(zbase + z * 128, 128), :],
                              gsem).wait()
      zrem = RPT_NODES % 128
      pltpu.sync_copy(rows.at[0].at[pl.ds(0, zrem), :],
                      agg_s.at[pl.ds(zbase + RPT_NODES - zrem, zrem), :])
      plsc.subcore_barrier()

      for sc_id in (0, 1):
        c = step if sc_id == 0 else half + step
        hc = h_hbm.at[c]

        @pl.when(cid == sc_id)
        def _():
          def stage(grow, half_i, n=G):
            sl = pl.ds(half_i * G, n)
            pltpu.sync_copy(src_hbm.at[pl.ds(grow, n), :], srcb.at[sl, :])
            pltpu.sync_copy(dst_hbm.at[pl.ds(grow, n), :], dstb.at[sl, :])

          def fire_gathers(half_i, n=G):
            for j in range(n):
              b = half_i * G + j
              pltpu.async_copy(hc.at[srcb.at[b]], rows.at[b], gsem)

          def drain_gathers(half_i, n=G):
            for j in range(n):
              b = half_i * G + j
              pltpu.make_async_copy(hc.at[srcb.at[b]], rows.at[b], gsem).wait()

          def fire_scatters(half_i, n=G):
            for j in range(n):
              b = half_i * G + j
              pltpu.async_copy(rows.at[b], agg_s.at[dstb.at[b]], ssem, add=True)

          def drain_scatters(half_i, n=G):
            for j in range(n):
              b = half_i * G + j
              pltpu.make_async_copy(rows.at[b], agg_s.at[dstb.at[b]], ssem,
                                    ).wait()

          def body(t, carry):
            grow = sid * ROWS_PER_TILE + t * (2 * G)
            stage(grow, 0)
            fire_gathers(0)
            stage(grow + G, 1)
            fire_gathers(1)
            drain_gathers(0)
            fire_scatters(0)
            drain_gathers(1)
            fire_scatters(1)
            drain_scatters(0)
            drain_scatters(1)
            return carry

          lax.fori_loop(0, N_OUTER, body, 0)
          tail = sid * ROWS_PER_TILE + N_OUTER * 2 * G
          stage(tail, 0, N_TAIL)
          fire_gathers(0, N_TAIL)
          drain_gathers(0, N_TAIL)
          fire_scatters(0, N_TAIL)
          drain_scatters(0, N_TAIL)

      plsc.subcore_barrier()

      for sc_id in (0, 1):
        c = step if sc_id == 0 else half + step

        @pl.when(cid == sc_id)
        def _():
          for z in range(RPT_NODES // ZROWS):
            sl = pl.ds(sid * RPT_NODES + z * ZROWS, ZROWS)
            pltpu.async_copy(agg_s.at[sl, :], out_hbm.at[sl, pl.ds(32 * c, 32)],
                             gsem)
          for z in range(RPT_NODES // ZROWS):
            sl = pl.ds(sid * RPT_NODES + z * ZROWS, ZROWS)
            pltpu.make_async_copy(agg_s.at[sl, :],
                                  out_hbm.at[sl, pl.ds(32 * c, 32)],
                                  gsem).wait()

      plsc.subcore_barrier()

  return k


_sc_agg2 = _make_sc_agg(2)
_sc_agg8 = _make_sc_agg(8)

ROW_BLK = 2000
N_BLKS = N // ROW_BLK


def _conv_body(x_ref, agg_ref, invdeg_ref, ws_ref, wn_ref, b_ref, h_ref, st_ref):
  if x_ref.shape[0] == 8 and len(x_ref.shape) == 3:
    x = jnp.concatenate([x_ref[c] for c in range(8)], axis=1)
  else:
    x = x_ref[...]
  hn = agg_ref[...] * invdeg_ref[...]
  h = (jnp.dot(x, ws_ref[...], preferred_element_type=jnp.float32)
       + jnp.dot(hn, wn_ref[...], preferred_element_type=jnp.float32)
       + b_ref[...])
  h_ref[...] = h

  @pl.when(pl.program_id(0) == 0)
  def _():
    st_ref[...] = jnp.zeros_like(st_ref)

  st_ref[0:1, :] = st_ref[0:1, :] + jnp.sum(h, axis=0, keepdims=True)
  st_ref[1:2, :] = st_ref[1:2, :] + jnp.sum(h * h, axis=0, keepdims=True)


def _conv_call(x, agg, invdeg, ws, wn, b):
  if x.ndim == 3:
    kin = HID
    x_spec = pl.BlockSpec((8, ROW_BLK, 32), lambda i: (0, i, 0))
  else:
    kin = x.shape[1]
    x_spec = pl.BlockSpec((ROW_BLK, kin), lambda i: (i, 0))
  return pl.pallas_call(
      _conv_body,
      grid=(N_BLKS,),
      in_specs=[
          x_spec,
          pl.BlockSpec((ROW_BLK, kin), lambda i: (i, 0)),
          pl.BlockSpec((ROW_BLK, 1), lambda i: (i, 0)),
          pl.BlockSpec((kin, HID), lambda i: (0, 0)),
          pl.BlockSpec((kin, HID), lambda i: (0, 0)),
          pl.BlockSpec((1, HID), lambda i: (0, 0)),
      ],
      out_specs=[
          pl.BlockSpec((ROW_BLK, HID), lambda i: (i, 0)),
          pl.BlockSpec((8, HID), lambda i: (0, 0)),
      ],
      out_shape=[
          jax.ShapeDtypeStruct((N, HID), jnp.float32),
          jax.ShapeDtypeStruct((8, HID), jnp.float32),
      ],
      compiler_params=pltpu.CompilerParams(
          dimension_semantics=("arbitrary",)),
  )(x, agg, invdeg, ws, wn, b.reshape(1, HID))


def _bn_body(st_ref, g_ref, bt_ref, h_ref, out_ref):
  mu = st_ref[0:1, :] * (1.0 / N)
  var = st_ref[1:2, :] * (1.0 / N) - mu * mu
  scale = g_ref[...] * lax.rsqrt(var + EPS)
  y = (h_ref[...] - mu) * scale + bt_ref[...]
  y = jnp.where(y >= 0, y, SLOPE * y)
  for c in range(8):
    out_ref[c, :, :] = y[:, 32 * c:32 * (c + 1)]


def _bn_call(h, st, gamma, beta):
  return pl.pallas_call(
      _bn_body,
      grid=(N_BLKS,),
      in_specs=[
          pl.BlockSpec((8, HID), lambda i: (0, 0)),
          pl.BlockSpec((1, HID), lambda i: (0, 0)),
          pl.BlockSpec((1, HID), lambda i: (0, 0)),
          pl.BlockSpec((ROW_BLK, HID), lambda i: (i, 0)),
      ],
      out_specs=pl.BlockSpec((8, ROW_BLK, 32), lambda i: (0, i, 0)),
      out_shape=jax.ShapeDtypeStruct((8, N, 32), jnp.float32),
      compiler_params=pltpu.CompilerParams(
          dimension_semantics=("arbitrary",)),
  )(st, gamma.reshape(1, HID), beta.reshape(1, HID), h)


def _bn_sum_body(st_ref, g_ref, bt_ref, h_ref, cs_ref):
  mu = st_ref[0:1, :] * (1.0 / N)
  var = st_ref[1:2, :] * (1.0 / N) - mu * mu
  scale = g_ref[...] * lax.rsqrt(var + EPS)
  y = (h_ref[...] - mu) * scale + bt_ref[...]
  y = jnp.where(y >= 0, y, SLOPE * y)

  @pl.when(pl.program_id(0) == 0)
  def _():
    cs_ref[...] = jnp.zeros_like(cs_ref)

  cs_ref[0:1, :] = cs_ref[0:1, :] + jnp.sum(y, axis=0, keepdims=True)


def _bn_sum_call(h, st, gamma, beta):
  return pl.pallas_call(
      _bn_sum_body,
      grid=(N_BLKS,),
      in_specs=[
          pl.BlockSpec((8, HID), lambda i: (0, 0)),
          pl.BlockSpec((1, HID), lambda i: (0, 0)),
          pl.BlockSpec((1, HID), lambda i: (0, 0)),
          pl.BlockSpec((ROW_BLK, HID), lambda i: (i, 0)),
      ],
      out_specs=pl.BlockSpec((8, HID), lambda i: (0, 0)),
      out_shape=jax.ShapeDtypeStruct((8, HID), jnp.float32),
      compiler_params=pltpu.CompilerParams(
          dimension_semantics=("arbitrary",)),
  )(st, gamma.reshape(1, HID), beta.reshape(1, HID), h)


def _head_body(cs_ref, w1_ref, b1_ref, w2_ref, b2_ref, w3_ref, b3_ref, out_ref):
  hg = cs_ref[0:1, :] * (1.0 / N)
  y = jnp.dot(hg, w1_ref[...], preferred_element_type=jnp.float32) + b1_ref[...]
  y = jnp.where(y >= 0, y, SLOPE * y)
  y = jnp.dot(y, w2_ref[...], preferred_element_type=jnp.float32) + b2_ref[...]
  y = jnp.where(y >= 0, y, SLOPE * y)
  y = jnp.dot(y, w3_ref[...], preferred_element_type=jnp.float32) + b3_ref[...]
  out_ref[...] = y


def _head_call(cs, w1, b1, w2, b2, w3p, b3p):
  return pl.pallas_call(
      _head_body,
      out_shape=jax.ShapeDtypeStruct((1, 128), jnp.float32),
  )(cs, w1, b1.reshape(1, HID), w2, b2.reshape(1, 1024), w3p, b3p)


def _chunked(h):
  f = h.shape[1]
  return h.reshape(N, f // 32, 32).transpose(1, 0, 2)


def kernel(x, edge_index, Ws1, Wn1, bc1, Ws2, Wn2, bc2, Ws3, Wn3, bc3,
           gamma1, beta1, gamma2, beta2, gamma3, beta3,
           fc1_W, fc1_b, fc2_W, fc2_b, fc3_W, fc3_b):
  src = edge_index[0]
  dst = edge_index[1]
  npad_e = EROWS * 128 - E
  src2d = jnp.concatenate([src, jnp.zeros((npad_e,), jnp.int32)]).reshape(EROWS, 128)
  dst2d = jnp.concatenate([dst, jnp.full((npad_e,), N, jnp.int32)]).reshape(EROWS, 128)

  x_pad = jnp.concatenate([x, jnp.ones((N, 1), jnp.float32)], axis=1)
  ws1p = jnp.zeros((64, HID), jnp.float32).at[:63].set(Ws1)
  wn1p = jnp.zeros((64, HID), jnp.float32).at[:63].set(Wn1)

  agg1 = _sc_agg2(_chunked(x_pad), src2d, dst2d)
  deg = agg1[:N, 63]
  invdeg = (1.0 / jnp.maximum(deg, 1.0)).reshape(N, 1)

  h1, st1 = _conv_call(x_pad, agg1, invdeg, ws1p, wn1p, bc1)
  h1c = _bn_call(h1, st1, gamma1, beta1)

  agg2 = _sc_agg8(h1c, src2d, dst2d)
  h2, st2 = _conv_call(h1c, agg2, invdeg, Ws2, Wn2, bc2)
  h2c = _bn_call(h2, st2, gamma2, beta2)

  agg3 = _sc_agg8(h2c, src2d, dst2d)
  h3, st3 = _conv_call(h2c, agg3, invdeg, Ws3, Wn3, bc3)
  cs3 = _bn_sum_call(h3, st3, gamma3, beta3)

  w3p = jnp.zeros((1024, 128), jnp.float32).at[:, :N_CLASSES].set(fc3_W)
  b3p = jnp.zeros((1, 128), jnp.float32).at[0, :N_CLASSES].set(fc3_b)
  y = _head_call(cs3, fc1_W, fc1_b, fc2_W, fc2_b, w3p, b3p)
  return y[:, :N_CLASSES]


# speedup vs baseline: 5.1416x; 1.1327x over previous
import functools

import jax
import jax.numpy as jnp
from jax import lax
from jax.experimental import pallas as pl
from jax.experimental.pallas import tpu as pltpu
from jax.experimental.pallas import tpu_sc as plsc

N = 50000
E = 800000
HID = 256
N_CLASSES = 18
EPS = 1e-5
SLOPE = 0.01

EROWS = 6272
EROWS_ALLOC = 6280
ROWS_PER_TILE = EROWS // 16
G = 3
N_OUTER = 65
N_TAIL = 2
NPAD = N + 8
NACC = 50048
RPT_NODES = NACC // 16
ZROWS = 184


def _make_sc_agg(n_chunks):
  half = n_chunks // 2
  mesh = plsc.VectorSubcoreMesh(core_axis_name="c", subcore_axis_name="s")

  @functools.partial(
      pl.kernel,
      mesh=mesh,
      out_type=jax.ShapeDtypeStruct((NACC, 32 * n_chunks), jnp.float32),
      scratch_types=[
          pltpu.VMEM_SHARED((NACC, 32), jnp.float32),
          pltpu.VMEM((2, 2 * G, 128), jnp.int32),
          pltpu.VMEM((2 * G, 128, 32), jnp.float32),
          pltpu.SemaphoreType.DMA,
          pltpu.SemaphoreType.DMA,
          pltpu.SemaphoreType.DMA,
      ],
      compiler_params=pltpu.CompilerParams(use_tc_tiling_on_sc=False),
  )
  def k(h_hbm, edges_hbm, out_hbm, agg_s, ebuf, rows, gsem, ssem, isem):
    cid = lax.axis_index("c")
    sid = lax.axis_index("s")

    for step in range(half):
      def zfill(i, carry):
        rows[0, i, pl.ds(0, 16)] = jnp.zeros((16,), jnp.float32)
        rows[0, i, pl.ds(16, 16)] = jnp.zeros((16,), jnp.float32)
        return carry

      lax.fori_loop(0, 128, zfill, 0)
      zbase = sid * RPT_NODES
      for z in range(RPT_NODES // 128):
        pltpu.async_copy(rows.at[0], agg_s.at[pl.ds(zbase + z * 128, 128), :],
                         gsem)
      for z in range(RPT_NODES // 128):
        pltpu.make_async_copy(rows.at[0],
                              agg_s.at[pl.ds(zbase + z * 128, 128), :],
                              gsem).wait()
      zrem = RPT_NODES % 128
      pltpu.sync_copy(rows.at[0].at[pl.ds(0, zrem), :],
                      agg_s.at[pl.ds(zbase + RPT_NODES - zrem, zrem), :])
      plsc.subcore_barrier()

      for sc_id in (0, 1):
        c = step if sc_id == 0 else half + step
        hc = h_hbm.at[c]

        @pl.when(cid == sc_id)
        def _():
          def stage_async(grow):
            pltpu.async_copy(edges_hbm.at[:, pl.ds(grow, 2 * G), :], ebuf, isem)

          def drain_stage(grow):
            pltpu.make_async_copy(edges_hbm.at[:, pl.ds(grow, 2 * G), :], ebuf,
                                  isem).wait()

          def fire_gathers(half_i, n=G):
            for j in range(n):
              b = half_i * G + j
              pltpu.async_copy(hc.at[ebuf.at[0, b]], rows.at[b], gsem)

          def drain_gathers(half_i, n=G):
            for j in range(n):
              b = half_i * G + j
              pltpu.make_async_copy(hc.at[ebuf.at[0, b]], rows.at[b],
                                    gsem).wait()

          def fire_scatters(half_i, n=G):
            for j in range(n):
              b = half_i * G + j
              pltpu.async_copy(rows.at[b], agg_s.at[ebuf.at[1, b]], ssem,
                               add=True)

          def drain_scatters(half_i, n=G):
            for j in range(n):
              b = half_i * G + j
              pltpu.make_async_copy(rows.at[b], agg_s.at[ebuf.at[1, b]], ssem,
                                    ).wait()

          base = sid * ROWS_PER_TILE
          stage_async(base)

          def body(t, carry):
            grow = base + t * (2 * G)
            drain_stage(grow)
            fire_gathers(0)
            fire_gathers(1)
            drain_gathers(0)
            fire_scatters(0)
            drain_gathers(1)
            fire_scatters(1)
            drain_scatters(0)
            drain_scatters(1)
            stage_async(grow + 2 * G)
            return carry

          lax.fori_loop(0, N_OUTER, body, 0)
          tail = base + N_OUTER * 2 * G
          drain_stage(tail)
          fire_gathers(0, N_TAIL)
          drain_gathers(0, N_TAIL)
          fire_scatters(0, N_TAIL)
          drain_scatters(0, N_TAIL)

      plsc.subcore_barrier()

      for sc_id in (0, 1):
        c = step if sc_id == 0 else half + step

        @pl.when(cid == sc_id)
        def _():
          for z in range(RPT_NODES // ZROWS):
            sl = pl.ds(sid * RPT_NODES + z * ZROWS, ZROWS)
            pltpu.async_copy(agg_s.at[sl, :], out_hbm.at[sl, pl.ds(32 * c, 32)],
                             gsem)
          for z in range(RPT_NODES // ZROWS):
            sl = pl.ds(sid * RPT_NODES + z * ZROWS, ZROWS)
            pltpu.make_async_copy(agg_s.at[sl, :],
                                  out_hbm.at[sl, pl.ds(32 * c, 32)],
                                  gsem).wait()

      plsc.subcore_barrier()

  return k


_sc_agg2 = _make_sc_agg(2)
_sc_agg8 = _make_sc_agg(8)

ROW_BLK = 2000
N_BLKS = N // ROW_BLK


def _conv_body(x_ref, agg_ref, invdeg_ref, ws_ref, wn_ref, b_ref, h_ref, st_ref):
  if x_ref.shape[0] == 8 and len(x_ref.shape) == 3:
    x = jnp.concatenate([x_ref[c] for c in range(8)], axis=1)
  else:
    x = x_ref[...]
  hn = agg_ref[...] * invdeg_ref[...]
  h = (jnp.dot(x, ws_ref[...], preferred_element_type=jnp.float32)
       + jnp.dot(hn, wn_ref[...], preferred_element_type=jnp.float32)
       + b_ref[...])
  h_ref[...] = h

  @pl.when(pl.program_id(0) == 0)
  def _():
    st_ref[...] = jnp.zeros_like(st_ref)

  st_ref[0:1, :] = st_ref[0:1, :] + jnp.sum(h, axis=0, keepdims=True)
  st_ref[1:2, :] = st_ref[1:2, :] + jnp.sum(h * h, axis=0, keepdims=True)


def _conv_call(x, agg, invdeg, ws, wn, b):
  if x.ndim == 3:
    kin = HID
    x_spec = pl.BlockSpec((8, ROW_BLK, 32), lambda i: (0, i, 0))
  else:
    kin = x.shape[1]
    x_spec = pl.BlockSpec((ROW_BLK, kin), lambda i: (i, 0))
  return pl.pallas_call(
      _conv_body,
      grid=(N_BLKS,),
      in_specs=[
          x_spec,
          pl.BlockSpec((ROW_BLK, kin), lambda i: (i, 0)),
          pl.BlockSpec((ROW_BLK, 1), lambda i: (i, 0)),
          pl.BlockSpec((kin, HID), lambda i: (0, 0)),
          pl.BlockSpec((kin, HID), lambda i: (0, 0)),
          pl.BlockSpec((1, HID), lambda i: (0, 0)),
      ],
      out_specs=[
          pl.BlockSpec((ROW_BLK, HID), lambda i: (i, 0)),
          pl.BlockSpec((8, HID), lambda i: (0, 0)),
      ],
      out_shape=[
          jax.ShapeDtypeStruct((N, HID), jnp.float32),
          jax.ShapeDtypeStruct((8, HID), jnp.float32),
      ],
      compiler_params=pltpu.CompilerParams(
          dimension_semantics=("arbitrary",)),
  )(x, agg, invdeg, ws, wn, b.reshape(1, HID))


def _bn_body(st_ref, g_ref, bt_ref, h_ref, out_ref):
  mu = st_ref[0:1, :] * (1.0 / N)
  var = st_ref[1:2, :] * (1.0 / N) - mu * mu
  scale = g_ref[...] * lax.rsqrt(var + EPS)
  y = (h_ref[...] - mu) * scale + bt_ref[...]
  y = jnp.where(y >= 0, y, SLOPE * y)
  for c in range(8):
    out_ref[c, :, :] = y[:, 32 * c:32 * (c + 1)]


def _bn_call(h, st, gamma, beta):
  return pl.pallas_call(
      _bn_body,
      grid=(N_BLKS,),
      in_specs=[
          pl.BlockSpec((8, HID), lambda i: (0, 0)),
          pl.BlockSpec((1, HID), lambda i: (0, 0)),
          pl.BlockSpec((1, HID), lambda i: (0, 0)),
          pl.BlockSpec((ROW_BLK, HID), lambda i: (i, 0)),
      ],
      out_specs=pl.BlockSpec((8, ROW_BLK, 32), lambda i: (0, i, 0)),
      out_shape=jax.ShapeDtypeStruct((8, N, 32), jnp.float32),
      compiler_params=pltpu.CompilerParams(
          dimension_semantics=("arbitrary",)),
  )(st, gamma.reshape(1, HID), beta.reshape(1, HID), h)


def _bn_sum_body(st_ref, g_ref, bt_ref, h_ref, cs_ref):
  mu = st_ref[0:1, :] * (1.0 / N)
  var = st_ref[1:2, :] * (1.0 / N) - mu * mu
  scale = g_ref[...] * lax.rsqrt(var + EPS)
  y = (h_ref[...] - mu) * scale + bt_ref[...]
  y = jnp.where(y >= 0, y, SLOPE * y)

  @pl.when(pl.program_id(0) == 0)
  def _():
    cs_ref[...] = jnp.zeros_like(cs_ref)

  cs_ref[0:1, :] = cs_ref[0:1, :] + jnp.sum(y, axis=0, keepdims=True)


def _bn_sum_call(h, st, gamma, beta):
  return pl.pallas_call(
      _bn_sum_body,
      grid=(N_BLKS,),
      in_specs=[
          pl.BlockSpec((8, HID), lambda i: (0, 0)),
          pl.BlockSpec((1, HID), lambda i: (0, 0)),
          pl.BlockSpec((1, HID), lambda i: (0, 0)),
          pl.BlockSpec((ROW_BLK, HID), lambda i: (i, 0)),
      ],
      out_specs=pl.BlockSpec((8, HID), lambda i: (0, 0)),
      out_shape=jax.ShapeDtypeStruct((8, HID), jnp.float32),
      compiler_params=pltpu.CompilerParams(
          dimension_semantics=("arbitrary",)),
  )(st, gamma.reshape(1, HID), beta.reshape(1, HID), h)


def _head_body(cs_ref, w1_ref, b1_ref, w2_ref, b2_ref, w3_ref, b3_ref, out_ref):
  hg = cs_ref[0:1, :] * (1.0 / N)
  y = jnp.dot(hg, w1_ref[...], preferred_element_type=jnp.float32) + b1_ref[...]
  y = jnp.where(y >= 0, y, SLOPE * y)
  y = jnp.dot(y, w2_ref[...], preferred_element_type=jnp.float32) + b2_ref[...]
  y = jnp.where(y >= 0, y, SLOPE * y)
  y = jnp.dot(y, w3_ref[...], preferred_element_type=jnp.float32) + b3_ref[...]
  out_ref[...] = y


def _head_call(cs, w1, b1, w2, b2, w3p, b3p):
  return pl.pallas_call(
      _head_body,
      out_shape=jax.ShapeDtypeStruct((1, 128), jnp.float32),
  )(cs, w1, b1.reshape(1, HID), w2, b2.reshape(1, 1024), w3p, b3p)


def _chunked(h):
  f = h.shape[1]
  return h.reshape(N, f // 32, 32).transpose(1, 0, 2)


def kernel(x, edge_index, Ws1, Wn1, bc1, Ws2, Wn2, bc2, Ws3, Wn3, bc3,
           gamma1, beta1, gamma2, beta2, gamma3, beta3,
           fc1_W, fc1_b, fc2_W, fc2_b, fc3_W, fc3_b):
  src = edge_index[0]
  dst = edge_index[1]
  npad_e = EROWS_ALLOC * 128 - E
  src2d = jnp.concatenate([src, jnp.zeros((npad_e,), jnp.int32)]
                          ).reshape(EROWS_ALLOC, 128)
  dst2d = jnp.concatenate([dst, jnp.full((npad_e,), N, jnp.int32)]
                          ).reshape(EROWS_ALLOC, 128)
  edges3d = jnp.stack([src2d, dst2d])

  x_pad = jnp.concatenate([x, jnp.ones((N, 1), jnp.float32)], axis=1)
  ws1p = jnp.zeros((64, HID), jnp.float32).at[:63].set(Ws1)
  wn1p = jnp.zeros((64, HID), jnp.float32).at[:63].set(Wn1)

  agg1 = _sc_agg2(_chunked(x_pad), edges3d)
  deg = agg1[:N, 63]
  invdeg = (1.0 / jnp.maximum(deg, 1.0)).reshape(N, 1)

  h1, st1 = _conv_call(x_pad, agg1, invdeg, ws1p, wn1p, bc1)
  h1c = _bn_call(h1, st1, gamma1, beta1)

  agg2 = _sc_agg8(h1c, edges3d)
  h2, st2 = _conv_call(h1c, agg2, invdeg, Ws2, Wn2, bc2)
  h2c = _bn_call(h2, st2, gamma2, beta2)

  agg3 = _sc_agg8(h2c, edges3d)
  h3, st3 = _conv_call(h2c, agg3, invdeg, Ws3, Wn3, bc3)
  cs3 = _bn_sum_call(h3, st3, gamma3, beta3)

  w3p = jnp.zeros((1024, 128), jnp.float32).at[:, :N_CLASSES].set(fc3_W)
  b3p = jnp.zeros((1, 128), jnp.float32).at[0, :N_CLASSES].set(fc3_b)
  y = _head_call(cs3, fc1_W, fc1_b, fc2_W, fc2_b, w3p, b3p)
  return y[:, :N_CLASSES]
